# Initial kernel scaffold; baseline (speedup 1.0000x reference)
import jax
import jax.numpy as jnp
from jax.experimental import pallas as pl


def kernel(tokens, table):
    raise NotImplementedError("write your pallas kernel here")


# speedup vs baseline: 4.7644x; 4.7644x over previous
import functools
import math

import jax
import jax.numpy as jnp
from jax import lax
from jax.experimental import pallas as pl
from jax.experimental.pallas import tpu as pltpu
from jax.experimental.pallas import tpu_sc as plsc


def _scale_body(table_ref, out_ref, *, scale):
    out_ref[...] = table_ref[...] * scale


@functools.lru_cache(maxsize=None)
def _make_gather(V, D, B_total):
    info = plsc.get_sparse_core_info()
    NW = info.num_cores * info.num_subcores
    assert B_total % NW == 0
    b_per_w = B_total // NW
    C = 128
    assert b_per_w % C == 0
    n_chunks = b_per_w // C
    mesh = plsc.VectorSubcoreMesh(core_axis_name="c", subcore_axis_name="s")

    @functools.partial(
        pl.kernel,
        mesh=mesh,
        out_type=jax.ShapeDtypeStruct((B_total, D), jnp.float32),
        scratch_types=[
            pltpu.VMEM((C,), jnp.int32),
            pltpu.VMEM((C, D), jnp.float32),
            pltpu.SemaphoreType.DMA,
        ],
    )
    def gather_kernel(idx_hbm, table_hbm, out_hbm, idx_v, rows_v, sem):
        wid = lax.axis_index("s") * info.num_cores + lax.axis_index("c")
        base = wid * b_per_w

        def body(i, carry):
            off = base + i * C
            pltpu.sync_copy(idx_hbm.at[pl.ds(off, C)], idx_v)
            pltpu.async_copy(table_hbm.at[idx_v], rows_v, sem).wait()
            pltpu.sync_copy(rows_v, out_hbm.at[pl.ds(off, C)])
            return carry

        lax.fori_loop(0, n_chunks, body, 0)

    return gather_kernel


def kernel(tokens, table):
    B, L = tokens.shape
    V, D = table.shape
    scale = math.sqrt(D)

    n_blocks = 50
    assert V % n_blocks == 0
    scaled = pl.pallas_call(
        functools.partial(_scale_body, scale=scale),
        out_shape=jax.ShapeDtypeStruct((V, D), jnp.float32),
        grid=(n_blocks,),
        in_specs=[pl.BlockSpec((V // n_blocks, D), lambda i: (i, 0))],
        out_specs=pl.BlockSpec((V // n_blocks, D), lambda i: (i, 0)),
    )(table)

    idx = tokens.reshape(-1).astype(jnp.int32)
    out = _make_gather(V, D, B * L)(idx, scaled)
    return out.reshape(B, L, D)
